# baseline (speedup 1.0000x reference)
import functools

import jax
import jax.numpy as jnp
from jax import lax
from jax.experimental import pallas as pl
from jax.experimental.pallas import tpu as pltpu
from jax.experimental.pallas import tpu_sc as plsc

_FS = 48000.0
_C = 343000.0
_N_MICS = 192
_OVERLAP = 2048
_WINDOW = 8192
_DK = _FS / (_C * 16.0)

_NC = 2
_MPW = 8
_NW = _N_MICS // _MPW
_WSTART = 1920
_MARGIN = _OVERLAP - _WSTART
_WLEN = _WINDOW + _MARGIN
_SPLIT = 4224
_NCHUNK = _WINDOW // 16
_HALF = _NCHUNK // 2


def _prep(pos, mic_pos):
    def body(pos_ref, mic_ref, o_ref):
        diff = mic_ref[...] - pos_ref[...]
        d = jnp.sqrt(jnp.sum(diff * diff, axis=1))
        delay = (jnp.max(d) - d) * _DK
        di = delay.astype(jnp.int32)
        di = jnp.minimum(jnp.maximum(di, 0), _MARGIN - 1)
        df = delay - di.astype(jnp.float32)
        rows = jnp.stack([
            di.astype(jnp.float32),
            df * (1.0 / _N_MICS),
            (1.0 - df) * (1.0 / _N_MICS),
        ])
        rows = jnp.pad(rows, ((0, 5), (0, 256 - _N_MICS)))
        o_ref[...] = rows

    return pl.pallas_call(
        body,
        out_shape=jax.ShapeDtypeStruct((8, 256), jnp.float32),
    )(pos, mic_pos)


def _sc_beamform(params, buf3):
    mesh = plsc.VectorSubcoreMesh(core_axis_name="c", subcore_axis_name="s")

    @functools.partial(
        pl.kernel,
        out_type=jax.ShapeDtypeStruct((_NW, _WINDOW), jnp.float32),
        mesh=mesh,
        compiler_params=pltpu.CompilerParams(needs_layout_passes=False),
        scratch_types=[
            pltpu.VMEM((8, 256), jnp.float32),
            pltpu.VMEM((_MPW * _WLEN,), jnp.float32),
            pltpu.VMEM((_WINDOW,), jnp.float32),
            pltpu.SemaphoreType.DMA,
            pltpu.SemaphoreType.DMA,
        ],
    )
    def sc_kernel(par_hbm, buf_hbm, out_hbm, par_v, rows_v, acc_v, sem1, sem2):
        wid = lax.axis_index("s") * _NC + lax.axis_index("c")

        @pl.when(wid < _NW)
        def _():
            cps1, cps2 = [], []
            for k in range(_MPW):
                cps1.append(pltpu.async_copy(
                    buf_hbm.at[wid, k, pl.ds(_WSTART, _SPLIT)],
                    rows_v.at[pl.ds(k * _WLEN, _SPLIT)], sem1))
            for k in range(_MPW):
                cps2.append(pltpu.async_copy(
                    buf_hbm.at[wid, k, pl.ds(_WSTART + _SPLIT, _WLEN - _SPLIT)],
                    rows_v.at[pl.ds(k * _WLEN + _SPLIT, _WLEN - _SPLIT)], sem2))

            pltpu.sync_copy(par_hbm, par_v)

            zeros16 = jnp.zeros((16,), jnp.int32)
            st1, s0, s1 = [], [], []
            for k in range(_MPW):
                midx = zeros16 + (wid * _MPW + k)
                di = plsc.load_gather(par_v, [zeros16, midx]).astype(jnp.int32)
                st1.append(k * _WLEN + _MARGIN - jnp.max(di))
                s0.append(plsc.load_gather(par_v, [zeros16 + 1, midx]))
                s1.append(plsc.load_gather(par_v, [zeros16 + 2, midx]))

            def chunk(i):
                off = i * 16
                acc = jnp.zeros((16,), jnp.float32)
                for k in range(_MPW):
                    x1 = rows_v[pl.ds(off + st1[k], 16)]
                    x0 = rows_v[pl.ds(off + st1[k] - 1, 16)]
                    acc = acc + x1 * s1[k] + x0 * s0[k]
                acc_v[pl.ds(off, 16)] = acc

            for cp in cps1:
                cp.wait()
            plsc.parallel_loop(0, _HALF, 1, unroll=4)(chunk)
            for cp in cps2:
                cp.wait()
            plsc.parallel_loop(_HALF, _NCHUNK, 1, unroll=4)(chunk)

            pltpu.sync_copy(acc_v, out_hbm.at[wid])

    return sc_kernel(params, buf3)


def _combine(parts):
    def body(x_ref, o_ref):
        o_ref[...] = jnp.sum(x_ref[...], axis=0)

    return pl.pallas_call(
        body,
        out_shape=jax.ShapeDtypeStruct((_WINDOW,), jnp.float32),
    )(parts)


def kernel(pos, buffer, mic_pos):
    params = _prep(pos, mic_pos)
    parts = _sc_beamform(params, buffer.reshape(_NW, _MPW, -1))
    return _combine(parts)


# speedup vs baseline: 1.0073x; 1.0073x over previous
import functools

import jax
import jax.numpy as jnp
from jax import lax
from jax.experimental import pallas as pl
from jax.experimental.pallas import tpu as pltpu
from jax.experimental.pallas import tpu_sc as plsc

_FS = 48000.0
_C = 343000.0
_N_MICS = 192
_OVERLAP = 2048
_WINDOW = 8192
_DK = _FS / (_C * 16.0)

_NC = 2
_MPW = 8
_NW = _N_MICS // _MPW
_WSTART = 1920
_MARGIN = _OVERLAP - _WSTART
_WLEN = _WINDOW + _MARGIN
_SPLIT = 4224
_NCHUNK = _WINDOW // 16
_HALF = _NCHUNK // 2


def _prep(pos, mic_pos):
    def body(pos_ref, mic_ref, o_ref):
        diff = mic_ref[...] - pos_ref[...]
        d = jnp.sqrt(jnp.sum(diff * diff, axis=1))
        delay = (jnp.max(d) - d) * _DK
        di = delay.astype(jnp.int32)
        di = jnp.minimum(jnp.maximum(di, 0), _MARGIN - 1)
        df = delay - di.astype(jnp.float32)
        rows = jnp.stack([
            di.astype(jnp.float32),
            df * (1.0 / _N_MICS),
            (1.0 - df) * (1.0 / _N_MICS),
        ])
        rows = jnp.pad(rows, ((0, 5), (0, 256 - _N_MICS)))
        o_ref[...] = rows

    return pl.pallas_call(
        body,
        out_shape=jax.ShapeDtypeStruct((8, 256), jnp.float32),
    )(pos, mic_pos)


def _sc_beamform(params, buf3):
    mesh = plsc.VectorSubcoreMesh(core_axis_name="c", subcore_axis_name="s")

    @functools.partial(
        pl.kernel,
        out_type=jax.ShapeDtypeStruct((_NW, _WINDOW), jnp.float32),
        mesh=mesh,
        compiler_params=pltpu.CompilerParams(needs_layout_passes=False),
        scratch_types=[
            pltpu.VMEM((8, 256), jnp.float32),
            pltpu.VMEM((_MPW * _WLEN,), jnp.float32),
            pltpu.VMEM((_WINDOW,), jnp.float32),
            pltpu.SemaphoreType.DMA,
            pltpu.SemaphoreType.DMA,
        ],
    )
    def sc_kernel(par_hbm, buf_hbm, out_hbm, par_v, rows_v, acc_v, sem1, sem2):
        wid = lax.axis_index("s") * _NC + lax.axis_index("c")

        @pl.when(wid < _NW)
        def _():
            cps1, cps2 = [], []
            for k in range(_MPW):
                cps1.append(pltpu.async_copy(
                    buf_hbm.at[wid, k, pl.ds(_WSTART, _SPLIT)],
                    rows_v.at[pl.ds(k * _WLEN, _SPLIT)], sem1))
            for k in range(_MPW):
                cps2.append(pltpu.async_copy(
                    buf_hbm.at[wid, k, pl.ds(_WSTART + _SPLIT, _WLEN - _SPLIT)],
                    rows_v.at[pl.ds(k * _WLEN + _SPLIT, _WLEN - _SPLIT)], sem2))

            pltpu.sync_copy(par_hbm, par_v)

            zeros16 = jnp.zeros((16,), jnp.int32)
            st1, s0, s1 = [], [], []
            for k in range(_MPW):
                midx = zeros16 + (wid * _MPW + k)
                di = plsc.load_gather(par_v, [zeros16, midx]).astype(jnp.int32)
                st1.append(k * _WLEN + _MARGIN - jnp.max(di))
                s0.append(plsc.load_gather(par_v, [zeros16 + 1, midx]))
                s1.append(plsc.load_gather(par_v, [zeros16 + 2, midx]))

            def chunk(i):
                off = i * 16
                acc = jnp.zeros((16,), jnp.float32)
                for k in range(_MPW):
                    x1 = rows_v[pl.ds(off + st1[k], 16)]
                    x0 = rows_v[pl.ds(off + st1[k] - 1, 16)]
                    acc = acc + x1 * s1[k] + x0 * s0[k]
                acc_v[pl.ds(off, 16)] = acc

            for cp in cps1:
                cp.wait()
            plsc.parallel_loop(0, _HALF, 1, unroll=2)(chunk)
            for cp in cps2:
                cp.wait()
            plsc.parallel_loop(_HALF, _NCHUNK, 1, unroll=2)(chunk)

            pltpu.sync_copy(acc_v, out_hbm.at[wid])

    return sc_kernel(params, buf3)


def _combine(parts):
    def body(x_ref, o_ref):
        o_ref[...] = jnp.sum(x_ref[...], axis=0)

    return pl.pallas_call(
        body,
        out_shape=jax.ShapeDtypeStruct((_WINDOW,), jnp.float32),
    )(parts)


def kernel(pos, buffer, mic_pos):
    params = _prep(pos, mic_pos)
    parts = _sc_beamform(params, buffer.reshape(_NW, _MPW, -1))
    return _combine(parts)
